# baseline (speedup 1.0000x reference)
import functools

import jax
import jax.numpy as jnp
from jax import lax
from jax.experimental import pallas as pl
from jax.experimental.pallas import tpu as pltpu
from jax.experimental.pallas import tpu_sc as plsc

_BS, _MC, _ML, _V, _P, _S = 16, 8, 20, 10000, 50, 300
_NTOK = _BS * _MC * _ML
_NROW = _BS * _MC
_NC, _NS = 2, 16
_NW = _NC * _NS
_TPW = _NTOK // _NW
_NCHUNK = _TPW // 16

_RB = 128
_NSTREAM = 4
_GRID = _NTOK // (_RB * _NSTREAM)


def _sc_gather_body(pos_tab, gt_pos, xpos_out, gtp_v, rowp_v, outp_v, semp):
    wid = lax.axis_index("s") * _NC + lax.axis_index("c")
    base = wid * _TPW
    pltpu.sync_copy(gt_pos.at[pl.ds(base, _TPW)], gtp_v)
    for i in range(_NCHUNK):
        sl = pl.ds(i * 16, 16)
        tok = lax.iota(jnp.int32, 16) + (base + i * 16)
        rowp_v[sl] = tok * _P + gtp_v[sl]
    pltpu.async_copy(pos_tab.at[rowp_v], outp_v, semp).wait()
    pltpu.sync_copy(outp_v, xpos_out.at[pl.ds(base, _TPW)])


@functools.cache
def _sc_gather_kernel():
  return functools.partial(
    pl.kernel,
    mesh=plsc.VectorSubcoreMesh(core_axis_name="c", subcore_axis_name="s",
                                num_cores=_NC, num_subcores=_NS),
    out_type=jax.ShapeDtypeStruct((_NTOK,), jnp.float32),
    scratch_types=[
        pltpu.VMEM((_TPW,), jnp.int32),
        pltpu.VMEM((_TPW,), jnp.int32),
        pltpu.VMEM((_TPW,), jnp.float32),
        pltpu.SemaphoreType.DMA,
    ],
  )(_sc_gather_body)


def _fused_body(*refs):
    caps = refs[0:_NSTREAM]
    poss = refs[_NSTREAM:2 * _NSTREAM]
    gtcs = refs[2 * _NSTREAM:3 * _NSTREAM]
    xposs = refs[3 * _NSTREAM:4 * _NSTREAM]
    lens_ref, cnt_ref, sem_x_ref, sem_y_ref = refs[4 * _NSTREAM:4 * _NSTREAM + 4]
    out_ref = refs[4 * _NSTREAM + 4]
    acc = refs[4 * _NSTREAM + 5]

    i = pl.program_id(0)

    @pl.when(i == 0)
    def _():
        acc[0] = 0.0
        acc[1] = 0.0
        acc[2] = 0.0
        acc[3] = 0.0

    cap_part = jnp.float32(0.0)
    pos_part = jnp.float32(0.0)
    ntok_part = jnp.float32(0.0)
    for k in range(_NSTREAM):
        x = caps[k][...]
        m = jnp.max(x, axis=1, keepdims=True)
        s = jnp.sum(jnp.exp(x - m), axis=1, keepdims=True)
        lse = jnp.log(s) + m
        gtc = gtcs[k][...]
        v = lax.broadcasted_iota(jnp.int32, (_RB, _V), 1)
        xcap = jnp.sum(jnp.where(v == gtc, x, 0.0), axis=1, keepdims=True)
        xp = poss[k][...]
        mp = jnp.max(xp, axis=1, keepdims=True)
        sp = jnp.sum(jnp.exp(xp - mp), axis=1, keepdims=True)
        lsep = jnp.log(sp) + mp
        gtp = xposs[k][...]
        vp = lax.broadcasted_iota(jnp.int32, (_RB, _P), 1)
        xpos = jnp.sum(jnp.where(vp == gtp, xp, 0.0), axis=1, keepdims=True)

        base = (k * _GRID + i) * _RB
        tok = lax.broadcasted_iota(jnp.int32, (_RB, 1), 0) + base
        bc = tok // _ML
        t = tok - bc * _ML
        b = tok // (_MC * _ML)
        jj = lax.broadcasted_iota(jnp.int32, (_RB, _NROW), 1)
        len_tok = jnp.sum(jnp.where(jj == bc, lens_ref[...], 0), axis=1,
                          keepdims=True)
        kk = lax.broadcasted_iota(jnp.int32, (_RB, _BS), 1)
        cnt_tok = jnp.sum(jnp.where(kk == b, cnt_ref[...], 0), axis=1,
                          keepdims=True)
        c_idx = bc - b * _MC
        tokf = ((t < len_tok) & (c_idx < cnt_tok)).astype(jnp.float32)
        cap_part += jnp.sum((lse - xcap) * tokf)
        pos_part += jnp.sum((lsep - xpos) * tokf)
        ntok_part += jnp.sum(tokf)

    acc[0] += cap_part
    acc[1] += pos_part
    acc[2] += ntok_part

    @pl.when(i == _GRID - 1)
    def _():
        xs = sem_x_ref[...]
        ys = sem_y_ref[...]
        bce = jnp.maximum(xs, 0.0) - xs * ys + jnp.log1p(jnp.exp(-jnp.abs(xs)))
        rr = lax.broadcasted_iota(jnp.int32, (_NROW, _BS), 1)
        bb = lax.broadcasted_iota(jnp.int32, (_NROW, _BS), 0) // _MC
        cnt_row = jnp.sum(jnp.where(rr == bb, cnt_ref[...], 0), axis=1,
                          keepdims=True)
        cc = lax.broadcasted_iota(jnp.int32, (_NROW, 1), 0) % _MC
        capf = (cc < cnt_row).astype(jnp.float32)
        sem_loss = jnp.sum(bce * capf) / (jnp.sum(capf) * _S)
        ntok = acc[2]
        cap_loss = acc[0] / ntok
        pos_loss = acc[1] / ntok
        out_ref[0] = cap_loss + sem_loss + pos_loss
        out_ref[1] = cap_loss
        out_ref[2] = sem_loss
        out_ref[3] = pos_loss


def _fused_call(cap2d, pos2d, gtc2d, xpos2d, lens, cnt, sem_x, sem_y):
    def vspec(s, width):
        return pl.BlockSpec((_RB, width), lambda i, s=s: (s * _GRID + i, 0))

    def wspec(shape):
        nd = len(shape)
        return pl.BlockSpec(shape, lambda i: (0,) * nd)

    return pl.pallas_call(
        _fused_body,
        grid=(_GRID,),
        in_specs=[vspec(s, _V) for s in range(_NSTREAM)]
                 + [vspec(s, _P) for s in range(_NSTREAM)]
                 + [vspec(s, 1) for s in range(_NSTREAM)]
                 + [vspec(s, 1) for s in range(_NSTREAM)]
                 + [wspec((1, _NROW)), wspec((1, _BS)),
                    wspec((_NROW, _S)), wspec((_NROW, _S))],
        out_specs=pl.BlockSpec(memory_space=pltpu.MemorySpace.SMEM),
        out_shape=jax.ShapeDtypeStruct((4,), jnp.float32),
        scratch_shapes=[pltpu.SMEM((4,), jnp.float32)],
    )(*([cap2d] * _NSTREAM + [pos2d] * _NSTREAM + [gtc2d] * _NSTREAM
        + [xpos2d] * _NSTREAM + [lens, cnt, sem_x, sem_y]))


def kernel(gt_captions, gt_cap_lens, pred_captions, gt_caps_sem_enc,
           pred_caps_sem_enc, gt_pos_seq, pred_pos_seq, gt_program,
           gt_prog_len, pred_program, gt_intervals, pred_intervals,
           gt_proposals, pred_proposals, gt_caps_count, pred_caps_count,
           gt_proposals_count):
    cap2d = pred_captions.reshape(_NTOK, _V)
    pos2d = pred_pos_seq.reshape(_NTOK, _P)
    pos_tab = pred_pos_seq.reshape(_NTOK * _P)
    gtc2d = gt_captions.reshape(_NTOK, 1).astype(jnp.int32)
    gt_posf = gt_pos_seq.reshape(_NTOK).astype(jnp.int32)

    out = _fused_call(
        cap2d, pos2d, gtc2d, gt_posf.reshape(_NTOK, 1),
        gt_cap_lens.reshape(1, _NROW).astype(jnp.int32),
        gt_caps_count.reshape(1, _BS).astype(jnp.int32),
        pred_caps_sem_enc.reshape(_NROW, _S),
        gt_caps_sem_enc.reshape(_NROW, _S),
    )
    return (out[0], out[1], out[2], out[3])


# speedup vs baseline: 1.0064x; 1.0064x over previous
import functools

import jax
import jax.numpy as jnp
from jax import lax
from jax.experimental import pallas as pl
from jax.experimental.pallas import tpu as pltpu
from jax.experimental.pallas import tpu_sc as plsc

_BS, _MC, _ML, _V, _P, _S = 16, 8, 20, 10000, 50, 300
_NTOK = _BS * _MC * _ML
_NROW = _BS * _MC
_NC, _NS = 2, 16
_NW = _NC * _NS
_TPW = _NTOK // _NW
_NCHUNK = _TPW // 16

_RB = 128
_NSTREAM = 4
_GRID = _NTOK // (_RB * _NSTREAM)


def _sc_gather_body(pos_tab, gt_pos, xpos_out, gtp_v, rowp_v, outp_v, semp):
    wid = lax.axis_index("s") * _NC + lax.axis_index("c")
    base = wid * _TPW
    pltpu.sync_copy(gt_pos.at[pl.ds(base, _TPW)], gtp_v)
    for i in range(_NCHUNK):
        sl = pl.ds(i * 16, 16)
        tok = lax.iota(jnp.int32, 16) + (base + i * 16)
        rowp_v[sl] = tok * _P + gtp_v[sl]
    pltpu.async_copy(pos_tab.at[rowp_v], outp_v, semp).wait()
    pltpu.sync_copy(outp_v, xpos_out.at[pl.ds(base, _TPW)])


@functools.cache
def _sc_gather_kernel():
  return functools.partial(
    pl.kernel,
    mesh=plsc.VectorSubcoreMesh(core_axis_name="c", subcore_axis_name="s",
                                num_cores=_NC, num_subcores=_NS),
    out_type=jax.ShapeDtypeStruct((_NTOK,), jnp.float32),
    scratch_types=[
        pltpu.VMEM((_TPW,), jnp.int32),
        pltpu.VMEM((_TPW,), jnp.int32),
        pltpu.VMEM((_TPW,), jnp.float32),
        pltpu.SemaphoreType.DMA,
    ],
  )(_sc_gather_body)


def _fused_body(*refs):
    caps = refs[0:_NSTREAM]
    poss = refs[_NSTREAM:2 * _NSTREAM]
    gtcs = refs[2 * _NSTREAM:3 * _NSTREAM]
    xposs = refs[3 * _NSTREAM:4 * _NSTREAM]
    lens_ref, cnt_ref, sem_x_ref, sem_y_ref = refs[4 * _NSTREAM:4 * _NSTREAM + 4]
    out_ref = refs[4 * _NSTREAM + 4]
    acc = refs[4 * _NSTREAM + 5]

    i = pl.program_id(0)

    @pl.when(i == 0)
    def _():
        acc[0] = 0.0
        acc[1] = 0.0
        acc[2] = 0.0
        acc[3] = 0.0

    cap_part = jnp.float32(0.0)
    pos_part = jnp.float32(0.0)
    ntok_part = jnp.float32(0.0)
    for k in range(_NSTREAM):
        x = caps[k][...]
        m = jnp.max(x, axis=1, keepdims=True)
        s = jnp.sum(jnp.exp(x - m), axis=1, keepdims=True)
        lse = jnp.log(s) + m
        gtc = gtcs[k][...]
        v = lax.broadcasted_iota(jnp.int32, (_RB, _V), 1)
        xcap = jnp.sum(jnp.where(v == gtc, x, 0.0), axis=1, keepdims=True)
        xp = poss[k][...]
        mp = jnp.max(xp, axis=1, keepdims=True)
        sp = jnp.sum(jnp.exp(xp - mp), axis=1, keepdims=True)
        lsep = jnp.log(sp) + mp
        xpos = xposs[k][...]

        base = (k * _GRID + i) * _RB
        tok = lax.broadcasted_iota(jnp.int32, (_RB, 1), 0) + base
        bc = tok // _ML
        t = tok - bc * _ML
        b = tok // (_MC * _ML)
        jj = lax.broadcasted_iota(jnp.int32, (_RB, _NROW), 1)
        len_tok = jnp.sum(jnp.where(jj == bc, lens_ref[...], 0), axis=1,
                          keepdims=True)
        kk = lax.broadcasted_iota(jnp.int32, (_RB, _BS), 1)
        cnt_tok = jnp.sum(jnp.where(kk == b, cnt_ref[...], 0), axis=1,
                          keepdims=True)
        c_idx = bc - b * _MC
        tokf = ((t < len_tok) & (c_idx < cnt_tok)).astype(jnp.float32)
        cap_part += jnp.sum((lse - xcap) * tokf)
        pos_part += jnp.sum((lsep - xpos) * tokf)
        ntok_part += jnp.sum(tokf)

    acc[0] += cap_part
    acc[1] += pos_part
    acc[2] += ntok_part

    @pl.when(i == _GRID - 1)
    def _():
        xs = sem_x_ref[...]
        ys = sem_y_ref[...]
        bce = jnp.maximum(xs, 0.0) - xs * ys + jnp.log1p(jnp.exp(-jnp.abs(xs)))
        rr = lax.broadcasted_iota(jnp.int32, (_NROW, _BS), 1)
        bb = lax.broadcasted_iota(jnp.int32, (_NROW, _BS), 0) // _MC
        cnt_row = jnp.sum(jnp.where(rr == bb, cnt_ref[...], 0), axis=1,
                          keepdims=True)
        cc = lax.broadcasted_iota(jnp.int32, (_NROW, 1), 0) % _MC
        capf = (cc < cnt_row).astype(jnp.float32)
        sem_loss = jnp.sum(bce * capf) / (jnp.sum(capf) * _S)
        ntok = acc[2]
        cap_loss = acc[0] / ntok
        pos_loss = acc[1] / ntok
        out_ref[0] = cap_loss + sem_loss + pos_loss
        out_ref[1] = cap_loss
        out_ref[2] = sem_loss
        out_ref[3] = pos_loss


def _fused_call(cap2d, pos2d, gtc2d, xpos2d, lens, cnt, sem_x, sem_y):
    def vspec(s, width):
        return pl.BlockSpec((_RB, width), lambda i, s=s: (s * _GRID + i, 0))

    def wspec(shape):
        nd = len(shape)
        return pl.BlockSpec(shape, lambda i: (0,) * nd)

    return pl.pallas_call(
        _fused_body,
        grid=(_GRID,),
        in_specs=[vspec(s, _V) for s in range(_NSTREAM)]
                 + [vspec(s, _P) for s in range(_NSTREAM)]
                 + [vspec(s, 1) for s in range(_NSTREAM)]
                 + [vspec(s, 1) for s in range(_NSTREAM)]
                 + [wspec((1, _NROW)), wspec((1, _BS)),
                    wspec((_NROW, _S)), wspec((_NROW, _S))],
        out_specs=pl.BlockSpec(memory_space=pltpu.MemorySpace.SMEM),
        out_shape=jax.ShapeDtypeStruct((4,), jnp.float32),
        scratch_shapes=[pltpu.SMEM((4,), jnp.float32)],
    )(*([cap2d] * _NSTREAM + [pos2d] * _NSTREAM + [gtc2d] * _NSTREAM
        + [xpos2d] * _NSTREAM + [lens, cnt, sem_x, sem_y]))


def kernel(gt_captions, gt_cap_lens, pred_captions, gt_caps_sem_enc,
           pred_caps_sem_enc, gt_pos_seq, pred_pos_seq, gt_program,
           gt_prog_len, pred_program, gt_intervals, pred_intervals,
           gt_proposals, pred_proposals, gt_caps_count, pred_caps_count,
           gt_proposals_count):
    cap2d = pred_captions.reshape(_NTOK, _V)
    pos2d = pred_pos_seq.reshape(_NTOK, _P)
    pos_tab = pred_pos_seq.reshape(_NTOK * _P)
    gtc2d = gt_captions.reshape(_NTOK, 1).astype(jnp.int32)
    gt_posf = gt_pos_seq.reshape(_NTOK).astype(jnp.int32)

    xpos = _sc_gather_kernel()(pos_tab, gt_posf)

    out = _fused_call(
        cap2d, pos2d, gtc2d, xpos.reshape(_NTOK, 1),
        gt_cap_lens.reshape(1, _NROW).astype(jnp.int32),
        gt_caps_count.reshape(1, _BS).astype(jnp.int32),
        pred_caps_sem_enc.reshape(_NROW, _S),
        gt_caps_sem_enc.reshape(_NROW, _S),
    )
    return (out[0], out[1], out[2], out[3])
